# baseline (speedup 1.0000x reference)
import functools

import jax
import jax.numpy as jnp
from jax.experimental import pallas as pl
from jax.experimental.pallas import tpu as pltpu

N = 10000
E = 160000
H = 128
NG = 50
NF = 128
S = 32
MH = 64
G = 16
NL = 2
NEMB = 10
K = int(0.25 * N)
V = 8
CUT = 10.0
TAU = 1.0
LAM = 0.1
LN2 = 0.6931471805599453

EB = 2048

_OFF_STEP = CUT / (NG - 1)
_COEFF = -0.5 / _OFF_STEP ** 2


def _ssp(x):
    return jnp.maximum(x, 0.0) + jnp.log1p(jnp.exp(-jnp.abs(x))) - LN2


def _smear_block(ew):
    g = jax.lax.broadcasted_iota(jnp.int32, (ew.shape[0], NG), 1).astype(jnp.float32)
    dd = ew[:, None] - g * _OFF_STEP
    return jnp.exp(_COEFF * dd * dd)


def _edge_w_body(scal_ref, w1_ref, b1_ref, w2_ref, b2_ref, out_ref):
    scal = scal_ref[...]
    ew = scal[:, 0]
    decay = scal[:, 1]
    valid = scal[:, 2]
    ea = _smear_block(ew) * decay[:, None]
    t = jnp.dot(ea, w1_ref[...], preferred_element_type=jnp.float32) + b1_ref[...]
    t = _ssp(t)
    w = jnp.dot(t, w2_ref[...], preferred_element_type=jnp.float32) + b2_ref[...]
    c = 0.5 * (jnp.cos(ew * (jnp.pi / CUT)) + 1.0)
    out_ref[...] = w * (c * valid)[:, None]


def _edge_w(ew, decay, valid, w1, b1, w2, b2):
    n = ew.shape[0]
    nb = -(-n // EB)
    npad = nb * EB
    scal = jnp.zeros((npad, 3), jnp.float32)
    scal = scal.at[:n, 0].set(ew).at[:n, 1].set(decay).at[:n, 2].set(valid)
    out = pl.pallas_call(
        _edge_w_body,
        grid=(nb,),
        in_specs=[
            pl.BlockSpec((EB, 3), lambda i: (i, 0)),
            pl.BlockSpec((NG, NF), lambda i: (0, 0)),
            pl.BlockSpec((1, NF), lambda i: (0, 0)),
            pl.BlockSpec((NF, NF), lambda i: (0, 0)),
            pl.BlockSpec((1, NF), lambda i: (0, 0)),
        ],
        out_specs=pl.BlockSpec((EB, NF), lambda i: (i, 0)),
        out_shape=jax.ShapeDtypeStruct((npad, NF), jnp.float32),
    )(scal, w1, b1.reshape(1, NF), w2, b2.reshape(1, NF))
    return out[:n]


def _edge_att_body(ew_ref, ab_ref, w1e_ref, w2_ref, out_ref):
    ew = ew_ref[...][:, 0]
    ea = _smear_block(ew)
    pre = ab_ref[...] + jnp.dot(ea, w1e_ref[...], preferred_element_type=jnp.float32)
    s = pre * (1.0 / (1.0 + jnp.exp(-pre)))
    out_ref[...] = jnp.dot(s, w2_ref[...], preferred_element_type=jnp.float32)


def _edge_att(ew, ab, w1e, w2):
    n = ew.shape[0]
    nb = -(-n // EB)
    npad = nb * EB
    ewp = jnp.zeros((npad, 1), jnp.float32).at[:n, 0].set(ew)
    abp = jnp.zeros((npad, H), jnp.float32).at[:n].set(ab)
    out = pl.pallas_call(
        _edge_att_body,
        grid=(nb,),
        in_specs=[
            pl.BlockSpec((EB, 1), lambda i: (i, 0)),
            pl.BlockSpec((EB, H), lambda i: (i, 0)),
            pl.BlockSpec((NG, H), lambda i: (0, 0)),
            pl.BlockSpec((H, 1), lambda i: (0, 0)),
        ],
        out_specs=pl.BlockSpec((EB, 1), lambda i: (i, 0)),
        out_shape=jax.ShapeDtypeStruct((npad, 1), jnp.float32),
    )(ewp, abp, w1e, w2)
    return out[:n, 0]


def _dist(p, q):
    d = p - q
    return jnp.sqrt(jnp.sum(d * d, -1) + 1e-9)


def _hmp(lp, h, pos, row, col, ew_full):
    hs = h[:, :S]
    sc = (jax.nn.relu(hs @ lp['msel_w1'] + lp['msel_b1']) @ lp['msel_w2'] + lp['msel_b2'])[:, 0]
    m = jax.nn.sigmoid(sc / TAU)
    _, midx = jax.lax.top_k(m, K)
    rank = jnp.full((N,), -1, jnp.int32).at[midx].set(jnp.arange(K, dtype=jnp.int32))
    rr = rank[row]
    rc = rank[col]
    vi = ((rr >= 0) & (rc >= 0)).astype(jnp.float32)
    ri = jnp.maximum(rr, 0) * (rc >= 0).astype(jnp.int32)
    ci = jnp.maximum(rc, 0) * (rr >= 0).astype(jnp.int32)
    hp = jnp.concatenate([h, pos], axis=1)[midx]
    hm = hp[:, :H]
    pm = hp[:, H:H + 3]

    att = LAM * ((hm[:, :S] @ lp['vgen_w']) @ hm[:, :S].T)
    adj = jnp.zeros((K, K), jnp.float32).at[ri, ci].add(vi)
    att = att - 1e30 * (adj > 0).astype(jnp.float32) - 1e30 * jnp.eye(K, dtype=jnp.float32)
    _, nbr = jax.lax.top_k(att, V)

    vr = jnp.repeat(jnp.arange(K, dtype=jnp.int32), V)
    vc = nbr.reshape(-1).astype(jnp.int32)

    a_n = hm @ lp['attn_w1'][:H]
    b_n = hm @ lp['attn_w1'][H:2 * H]
    x = hm @ lp['lin1_w']
    tab = jnp.concatenate([a_n, b_n, x, pm], axis=1)
    tv = tab[vc]
    a_vc = tv[:, :H]
    b_vc = tv[:, H:2 * H]
    x_vc = tv[:, 2 * H:3 * H]
    pm_vc = tv[:, 3 * H:3 * H + 3]

    a_rep = jnp.repeat(a_n, V, axis=0)
    b_rep = jnp.repeat(b_n, V, axis=0)
    pm_rep = jnp.repeat(pm, V, axis=0)
    ew_h = _dist(pm_rep, pm_vc)
    ew_v = jnp.concatenate([ew_h, ew_h])

    ab = jnp.concatenate([a_rep + b_vc, a_vc + b_rep]) + lp['attn_b1']
    s_v = _edge_att(ew_v, ab, lp['attn_w1'][2 * H:], lp['attn_w2']) + lp['attn_b2'][0]

    s1 = s_v[:K * V].reshape(K, V)
    s2 = s_v[K * V:]
    mx = jnp.maximum(jnp.max(s1, axis=1), jax.ops.segment_max(s2, vc, num_segments=K))
    mx = jax.lax.stop_gradient(mx)
    ex1 = jnp.exp(s1 - mx[:, None])
    ex2 = jnp.exp(s2 - mx[vc])
    den = jnp.sum(ex1, axis=1) + jax.ops.segment_sum(ex2, vc, num_segments=K)
    dec1 = ex1 / (den[:, None] + 1e-12)
    dec2 = ex2 / (den[vc] + 1e-12)
    dec_v = jnp.concatenate([dec1.reshape(-1), dec2])

    rowm = jnp.concatenate([ri, vr, vc])
    colm = jnp.concatenate([ci, vc, vr])
    ew_m = jnp.concatenate([ew_full, ew_v])
    decay = jnp.concatenate([jnp.ones((E,), jnp.float32), dec_v])
    validm = jnp.concatenate([vi, jnp.ones((2 * K * V,), jnp.float32)])
    w_m = _edge_w(ew_m, decay, validm, lp['mlp_w1'], lp['mlp_b1'], lp['mlp_w2'], lp['mlp_b2'])
    x_rows = jnp.concatenate([x[ri], jnp.repeat(x, V, axis=0), x_vc])
    agg = jnp.zeros((K, NF), jnp.float32).at[colm].add(x_rows * w_m)
    hup = _ssp(agg @ lp['lin2_w'] + lp['lin2_b']) @ lp['lin_w'] + lp['lin_b']
    hh = hm + hup
    hexp = jnp.zeros_like(h).at[midx].set(hh)
    hf = (1.0 - m[:, None]) * h + m[:, None] * hexp

    ones_e = jnp.ones((E,), jnp.float32)
    w_f = _edge_w(ew_full, ones_e, ones_e, lp['mlp_w1'], lp['mlp_b1'], lp['mlp_w2'], lp['mlp_b2'])
    x2 = hf @ lp['lin1_w']
    agg2 = jnp.zeros((N, NF), jnp.float32).at[col].add(x2[row] * w_f)
    hup2 = _ssp(agg2 @ lp['lin2_w'] + lp['lin2_b']) @ lp['lin_w'] + lp['lin_b']
    return hf + hup2


def kernel(atoms, pos, batch, edge_index, params):
    row = edge_index[0]
    col = edge_index[1]
    h = params['emb'][atoms]
    pc = pos[jnp.concatenate([row, col])]
    ew_full = _dist(pc[:E], pc[E:])
    for lp in params['layers']:
        h = _hmp(lp, h, pos, row, col, ew_full)
    h2 = jax.nn.silu(h @ params['trunk_w'] + params['trunk_b'])
    e_atom = (h2 @ params['e_w'] + params['e_b'])[:, 0]
    q_atom = (h2 @ params['q_w'] + params['q_b'])[:, 0]
    e_tot = jax.ops.segment_sum(e_atom, batch, num_segments=G)
    q_tot = jax.ops.segment_sum(q_atom, batch, num_segments=G)
    return (e_atom, q_atom, e_tot, q_tot)


# speedup vs baseline: 1.1338x; 1.1338x over previous
import functools

import jax
import jax.numpy as jnp
from jax.experimental import pallas as pl
from jax.experimental.pallas import tpu as pltpu

N = 10000
E = 160000
H = 128
NG = 50
NF = 128
S = 32
MH = 64
G = 16
NL = 2
NEMB = 10
K = int(0.25 * N)
V = 8
CUT = 10.0
TAU = 1.0
LAM = 0.1
LN2 = 0.6931471805599453

EB = 2048

_OFF_STEP = CUT / (NG - 1)
_COEFF = -0.5 / _OFF_STEP ** 2


def _ssp(x):
    return jnp.maximum(x, 0.0) + jnp.log1p(jnp.exp(-jnp.abs(x))) - LN2


def _smear_block(ew):
    g = jax.lax.broadcasted_iota(jnp.int32, (ew.shape[0], NG), 1).astype(jnp.float32)
    dd = ew[:, None] - g * _OFF_STEP
    return jnp.exp(_COEFF * dd * dd)


def _edge_w_body(scal_ref, w1_ref, b1_ref, w2_ref, b2_ref, out_ref):
    scal = scal_ref[...]
    ew = scal[:, 0]
    decay = scal[:, 1]
    valid = scal[:, 2]
    ea = _smear_block(ew) * decay[:, None]
    t = jnp.dot(ea, w1_ref[...], preferred_element_type=jnp.float32) + b1_ref[...]
    t = _ssp(t)
    w = jnp.dot(t, w2_ref[...], preferred_element_type=jnp.float32) + b2_ref[...]
    c = 0.5 * (jnp.cos(ew * (jnp.pi / CUT)) + 1.0)
    out_ref[...] = w * (c * valid)[:, None]


def _edge_w(ew, decay, valid, w1, b1, w2, b2):
    n = ew.shape[0]
    nb = -(-n // EB)
    npad = nb * EB
    scal = jnp.zeros((npad, 3), jnp.float32)
    scal = scal.at[:n, 0].set(ew).at[:n, 1].set(decay).at[:n, 2].set(valid)
    out = pl.pallas_call(
        _edge_w_body,
        grid=(nb,),
        in_specs=[
            pl.BlockSpec((EB, 3), lambda i: (i, 0)),
            pl.BlockSpec((NG, NF), lambda i: (0, 0)),
            pl.BlockSpec((1, NF), lambda i: (0, 0)),
            pl.BlockSpec((NF, NF), lambda i: (0, 0)),
            pl.BlockSpec((1, NF), lambda i: (0, 0)),
        ],
        out_specs=pl.BlockSpec((EB, NF), lambda i: (i, 0)),
        out_shape=jax.ShapeDtypeStruct((npad, NF), jnp.float32),
    )(scal, w1, b1.reshape(1, NF), w2, b2.reshape(1, NF))
    return out[:n]


def _edge_att_body(ew_ref, ab_ref, w1e_ref, w2_ref, out_ref):
    ew = ew_ref[...][:, 0]
    ea = _smear_block(ew)
    pre = ab_ref[...] + jnp.dot(ea, w1e_ref[...], preferred_element_type=jnp.float32)
    s = pre * (1.0 / (1.0 + jnp.exp(-pre)))
    out_ref[...] = jnp.dot(s, w2_ref[...], preferred_element_type=jnp.float32)


def _edge_att(ew, ab, w1e, w2):
    n = ew.shape[0]
    nb = -(-n // EB)
    npad = nb * EB
    ewp = jnp.zeros((npad, 1), jnp.float32).at[:n, 0].set(ew)
    abp = jnp.zeros((npad, H), jnp.float32).at[:n].set(ab)
    out = pl.pallas_call(
        _edge_att_body,
        grid=(nb,),
        in_specs=[
            pl.BlockSpec((EB, 1), lambda i: (i, 0)),
            pl.BlockSpec((EB, H), lambda i: (i, 0)),
            pl.BlockSpec((NG, H), lambda i: (0, 0)),
            pl.BlockSpec((H, 1), lambda i: (0, 0)),
        ],
        out_specs=pl.BlockSpec((EB, 1), lambda i: (i, 0)),
        out_shape=jax.ShapeDtypeStruct((npad, 1), jnp.float32),
    )(ewp, abp, w1e, w2)
    return out[:n, 0]


RB = 256
KP = 2560


def _top8_body(hs2_ref, hmt_ref, adj_ref, out_ref):
    r0 = pl.program_id(0) * RB
    att = LAM * jnp.dot(hs2_ref[...], hmt_ref[...], preferred_element_type=jnp.float32)
    cols = jax.lax.broadcasted_iota(jnp.int32, (RB, KP), 1)
    rows = jax.lax.broadcasted_iota(jnp.int32, (RB, KP), 0) + r0
    att = att - 1e30 * (adj_ref[...] > 0).astype(jnp.float32)
    att = jnp.where(cols == rows, att - 1e30, att)
    att = jnp.where(cols >= K, -3.0e30, att)
    picks = []
    for _ in range(V):
        mx = jnp.max(att, axis=1)
        sel = jnp.min(jnp.where(att == mx[:, None], cols, KP), axis=1)
        picks.append(sel)
        att = jnp.where(cols == sel[:, None], -3.5e30, att)
    out_ref[...] = jnp.stack(picks, axis=1)


def _top8(hs2, hmt, adj):
    return pl.pallas_call(
        _top8_body,
        grid=(KP // RB,),
        in_specs=[
            pl.BlockSpec((RB, S), lambda i: (i, 0)),
            pl.BlockSpec((S, KP), lambda i: (0, 0)),
            pl.BlockSpec((RB, KP), lambda i: (i, 0)),
        ],
        out_specs=pl.BlockSpec((RB, V), lambda i: (i, 0)),
        out_shape=jax.ShapeDtypeStruct((KP, V), jnp.int32),
    )(hs2, hmt, adj)[:K]


def _dist(p, q):
    d = p - q
    return jnp.sqrt(jnp.sum(d * d, -1) + 1e-9)


def _hmp(lp, h, pos, row, col, ew_full):
    hs = h[:, :S]
    sc = (jax.nn.relu(hs @ lp['msel_w1'] + lp['msel_b1']) @ lp['msel_w2'] + lp['msel_b2'])[:, 0]
    m = jax.nn.sigmoid(sc / TAU)
    _, midx = jax.lax.top_k(m, K)
    rank = jnp.full((N,), -1, jnp.int32).at[midx].set(jnp.arange(K, dtype=jnp.int32))
    rr = rank[row]
    rc = rank[col]
    vi = ((rr >= 0) & (rc >= 0)).astype(jnp.float32)
    ri = jnp.maximum(rr, 0) * (rc >= 0).astype(jnp.int32)
    ci = jnp.maximum(rc, 0) * (rr >= 0).astype(jnp.int32)
    hp = jnp.concatenate([h, pos], axis=1)[midx]
    hm = hp[:, :H]
    pm = hp[:, H:H + 3]

    hs2 = jnp.zeros((KP, S), jnp.float32).at[:K].set(hm[:, :S] @ lp['vgen_w'])
    hmt = jnp.zeros((S, KP), jnp.float32).at[:, :K].set(hm[:, :S].T)
    adj = jnp.zeros((KP, KP), jnp.float32).at[ri, ci].add(vi)
    nbr = _top8(hs2, hmt, adj)

    vr = jnp.repeat(jnp.arange(K, dtype=jnp.int32), V)
    vc = nbr.reshape(-1).astype(jnp.int32)

    a_n = hm @ lp['attn_w1'][:H]
    b_n = hm @ lp['attn_w1'][H:2 * H]
    x = hm @ lp['lin1_w']
    tab = jnp.concatenate([a_n, b_n, x, pm], axis=1)
    tv = tab[vc]
    a_vc = tv[:, :H]
    b_vc = tv[:, H:2 * H]
    x_vc = tv[:, 2 * H:3 * H]
    pm_vc = tv[:, 3 * H:3 * H + 3]

    a_rep = jnp.repeat(a_n, V, axis=0)
    b_rep = jnp.repeat(b_n, V, axis=0)
    pm_rep = jnp.repeat(pm, V, axis=0)
    ew_h = _dist(pm_rep, pm_vc)
    ew_v = jnp.concatenate([ew_h, ew_h])

    ab = jnp.concatenate([a_rep + b_vc, a_vc + b_rep]) + lp['attn_b1']
    s_v = _edge_att(ew_v, ab, lp['attn_w1'][2 * H:], lp['attn_w2']) + lp['attn_b2'][0]

    s1 = s_v[:K * V].reshape(K, V)
    s2 = s_v[K * V:]
    mx = jnp.maximum(jnp.max(s1, axis=1), jax.ops.segment_max(s2, vc, num_segments=K))
    mx = jax.lax.stop_gradient(mx)
    ex1 = jnp.exp(s1 - mx[:, None])
    ex2 = jnp.exp(s2 - mx[vc])
    den = jnp.sum(ex1, axis=1) + jax.ops.segment_sum(ex2, vc, num_segments=K)
    dec1 = ex1 / (den[:, None] + 1e-12)
    dec2 = ex2 / (den[vc] + 1e-12)
    dec_v = jnp.concatenate([dec1.reshape(-1), dec2])

    rowm = jnp.concatenate([ri, vr, vc])
    colm = jnp.concatenate([ci, vc, vr])
    ew_m = jnp.concatenate([ew_full, ew_v])
    decay = jnp.concatenate([jnp.ones((E,), jnp.float32), dec_v])
    validm = jnp.concatenate([vi, jnp.ones((2 * K * V,), jnp.float32)])
    w_m = _edge_w(ew_m, decay, validm, lp['mlp_w1'], lp['mlp_b1'], lp['mlp_w2'], lp['mlp_b2'])
    x_rows = jnp.concatenate([x[ri], jnp.repeat(x, V, axis=0), x_vc])
    agg = jnp.zeros((K, NF), jnp.float32).at[colm].add(x_rows * w_m)
    hup = _ssp(agg @ lp['lin2_w'] + lp['lin2_b']) @ lp['lin_w'] + lp['lin_b']
    hh = hm + hup
    hexp = jnp.zeros_like(h).at[midx].set(hh)
    hf = (1.0 - m[:, None]) * h + m[:, None] * hexp

    ones_e = jnp.ones((E,), jnp.float32)
    w_f = _edge_w(ew_full, ones_e, ones_e, lp['mlp_w1'], lp['mlp_b1'], lp['mlp_w2'], lp['mlp_b2'])
    x2 = hf @ lp['lin1_w']
    agg2 = jnp.zeros((N, NF), jnp.float32).at[col].add(x2[row] * w_f)
    hup2 = _ssp(agg2 @ lp['lin2_w'] + lp['lin2_b']) @ lp['lin_w'] + lp['lin_b']
    return hf + hup2


def kernel(atoms, pos, batch, edge_index, params):
    row = edge_index[0]
    col = edge_index[1]
    h = params['emb'][atoms]
    pc = pos[jnp.concatenate([row, col])]
    ew_full = _dist(pc[:E], pc[E:])
    for lp in params['layers']:
        h = _hmp(lp, h, pos, row, col, ew_full)
    h2 = jax.nn.silu(h @ params['trunk_w'] + params['trunk_b'])
    e_atom = (h2 @ params['e_w'] + params['e_b'])[:, 0]
    q_atom = (h2 @ params['q_w'] + params['q_b'])[:, 0]
    e_tot = jax.ops.segment_sum(e_atom, batch, num_segments=G)
    q_tot = jax.ops.segment_sum(q_atom, batch, num_segments=G)
    return (e_atom, q_atom, e_tot, q_tot)


# speedup vs baseline: 1.6224x; 1.4310x over previous
import functools

import jax
import jax.numpy as jnp
from jax import lax
from jax.experimental import pallas as pl
from jax.experimental.pallas import tpu as pltpu
from jax.experimental.pallas import tpu_sc as plsc

N = 10000
E = 160000
H = 128
NG = 50
NF = 128
S = 32
MH = 64
G = 16
NL = 2
NEMB = 10
K = int(0.25 * N)
V = 8
CUT = 10.0
TAU = 1.0
LAM = 0.1
LN2 = 0.6931471805599453

EB = 2048

_OFF_STEP = CUT / (NG - 1)
_COEFF = -0.5 / _OFF_STEP ** 2


def _ssp(x):
    return jnp.maximum(x, 0.0) + jnp.log1p(jnp.exp(-jnp.abs(x))) - LN2


def _smear_block(ew):
    g = jax.lax.broadcasted_iota(jnp.int32, (ew.shape[0], NG), 1).astype(jnp.float32)
    dd = ew[:, None] - g * _OFF_STEP
    return jnp.exp(_COEFF * dd * dd)


def _edge_w_body(scal_ref, w1_ref, b1_ref, w2_ref, b2_ref, out_ref):
    scal = scal_ref[...]
    ew = scal[:, 0]
    decay = scal[:, 1]
    valid = scal[:, 2]
    ea = _smear_block(ew) * decay[:, None]
    t = jnp.dot(ea, w1_ref[...], preferred_element_type=jnp.float32) + b1_ref[...]
    t = _ssp(t)
    w = jnp.dot(t, w2_ref[...], preferred_element_type=jnp.float32) + b2_ref[...]
    c = 0.5 * (jnp.cos(ew * (jnp.pi / CUT)) + 1.0)
    out_ref[...] = w * (c * valid)[:, None]


def _edge_w(ew, decay, valid, w1, b1, w2, b2):
    n = ew.shape[0]
    nb = -(-n // EB)
    npad = nb * EB
    scal = jnp.zeros((npad, 3), jnp.float32)
    scal = scal.at[:n, 0].set(ew).at[:n, 1].set(decay).at[:n, 2].set(valid)
    out = pl.pallas_call(
        _edge_w_body,
        grid=(nb,),
        in_specs=[
            pl.BlockSpec((EB, 3), lambda i: (i, 0)),
            pl.BlockSpec((NG, NF), lambda i: (0, 0)),
            pl.BlockSpec((1, NF), lambda i: (0, 0)),
            pl.BlockSpec((NF, NF), lambda i: (0, 0)),
            pl.BlockSpec((1, NF), lambda i: (0, 0)),
        ],
        out_specs=pl.BlockSpec((EB, NF), lambda i: (i, 0)),
        out_shape=jax.ShapeDtypeStruct((npad, NF), jnp.float32),
    )(scal, w1, b1.reshape(1, NF), w2, b2.reshape(1, NF))
    return out[:n]


def _edge_att_body(ew_ref, ab_ref, w1e_ref, w2_ref, out_ref):
    ew = ew_ref[...][:, 0]
    ea = _smear_block(ew)
    pre = ab_ref[...] + jnp.dot(ea, w1e_ref[...], preferred_element_type=jnp.float32)
    s = pre * (1.0 / (1.0 + jnp.exp(-pre)))
    out_ref[...] = jnp.dot(s, w2_ref[...], preferred_element_type=jnp.float32)


def _edge_att(ew, ab, w1e, w2):
    n = ew.shape[0]
    nb = -(-n // EB)
    npad = nb * EB
    ewp = jnp.zeros((npad, 1), jnp.float32).at[:n, 0].set(ew)
    abp = jnp.zeros((npad, H), jnp.float32).at[:n].set(ab)
    out = pl.pallas_call(
        _edge_att_body,
        grid=(nb,),
        in_specs=[
            pl.BlockSpec((EB, 1), lambda i: (i, 0)),
            pl.BlockSpec((EB, H), lambda i: (i, 0)),
            pl.BlockSpec((NG, H), lambda i: (0, 0)),
            pl.BlockSpec((H, 1), lambda i: (0, 0)),
        ],
        out_specs=pl.BlockSpec((EB, 1), lambda i: (i, 0)),
        out_shape=jax.ShapeDtypeStruct((npad, 1), jnp.float32),
    )(ewp, abp, w1e, w2)
    return out[:n, 0]


NW = 32
EPW = 5120
EPAD = NW * EPW


def _scidx_body(rank_hbm, row_hbm, col_hbm, vi_hbm, ri_hbm, ci_hbm,
                rank_v, row_v, col_v, vi_v, ri_v, ci_v):
    wid = lax.axis_index("s") * 2 + lax.axis_index("c")
    base = wid * EPW
    pltpu.sync_copy(rank_hbm, rank_v)
    pltpu.sync_copy(row_hbm.at[pl.ds(base, EPW)], row_v)
    pltpu.sync_copy(col_hbm.at[pl.ds(base, EPW)], col_v)

    def body(j, carry):
        o = j * 16
        rr = plsc.load_gather(rank_v, [row_v[pl.ds(o, 16)]])
        cc = plsc.load_gather(rank_v, [col_v[pl.ds(o, 16)]])
        both = (rr >= 0) & (cc >= 0)
        vi_v[pl.ds(o, 16)] = jnp.where(both, 1.0, 0.0)
        ri_v[pl.ds(o, 16)] = jnp.where(both, rr, 0)
        ci_v[pl.ds(o, 16)] = jnp.where(both, cc, 0)
        return carry

    lax.fori_loop(0, EPW // 16, body, 0, unroll=8)
    pltpu.sync_copy(vi_v, vi_hbm.at[pl.ds(base, EPW)])
    pltpu.sync_copy(ri_v, ri_hbm.at[pl.ds(base, EPW)])
    pltpu.sync_copy(ci_v, ci_hbm.at[pl.ds(base, EPW)])


def _sc_edge_index(rank, row_p, col_p):
    f = pl.kernel(
        _scidx_body,
        out_type=[
            jax.ShapeDtypeStruct((EPAD,), jnp.float32),
            jax.ShapeDtypeStruct((EPAD,), jnp.int32),
            jax.ShapeDtypeStruct((EPAD,), jnp.int32),
        ],
        mesh=plsc.VectorSubcoreMesh(core_axis_name="c", subcore_axis_name="s"),
        compiler_params=pltpu.CompilerParams(needs_layout_passes=False),
        scratch_types=[
            pltpu.VMEM((N,), jnp.int32),
            pltpu.VMEM((EPW,), jnp.int32),
            pltpu.VMEM((EPW,), jnp.int32),
            pltpu.VMEM((EPW,), jnp.float32),
            pltpu.VMEM((EPW,), jnp.int32),
            pltpu.VMEM((EPW,), jnp.int32),
        ],
    )
    return f(rank, row_p, col_p)


RB = 256
KP = 2560


def _top8_body(hs2_ref, hmt_ref, adj_ref, out_ref):
    r0 = pl.program_id(0) * RB
    att = LAM * jnp.dot(hs2_ref[...], hmt_ref[...], preferred_element_type=jnp.float32)
    cols = jax.lax.broadcasted_iota(jnp.int32, (RB, KP), 1)
    rows = jax.lax.broadcasted_iota(jnp.int32, (RB, KP), 0) + r0
    att = att - 1e30 * (adj_ref[...] > 0).astype(jnp.float32)
    att = jnp.where(cols == rows, att - 1e30, att)
    att = jnp.where(cols >= K, -3.0e30, att)
    picks = []
    for _ in range(V):
        mx = jnp.max(att, axis=1)
        sel = jnp.min(jnp.where(att == mx[:, None], cols, KP), axis=1)
        picks.append(sel)
        att = jnp.where(cols == sel[:, None], -3.5e30, att)
    out_ref[...] = jnp.stack(picks, axis=1)


def _top8(hs2, hmt, adj):
    return pl.pallas_call(
        _top8_body,
        grid=(KP // RB,),
        in_specs=[
            pl.BlockSpec((RB, S), lambda i: (i, 0)),
            pl.BlockSpec((S, KP), lambda i: (0, 0)),
            pl.BlockSpec((RB, KP), lambda i: (i, 0)),
        ],
        out_specs=pl.BlockSpec((RB, V), lambda i: (i, 0)),
        out_shape=jax.ShapeDtypeStruct((KP, V), jnp.int32),
    )(hs2, hmt, adj)[:K]


def _dist(p, q):
    d = p - q
    return jnp.sqrt(jnp.sum(d * d, -1) + 1e-9)


def _hmp(lp, h, pos, row, col, ew_full, row_p, col_p):
    hs = h[:, :S]
    sc = (jax.nn.relu(hs @ lp['msel_w1'] + lp['msel_b1']) @ lp['msel_w2'] + lp['msel_b2'])[:, 0]
    m = jax.nn.sigmoid(sc / TAU)
    _, midx = jax.lax.top_k(m, K)
    rank = jnp.full((N,), -1, jnp.int32).at[midx].set(jnp.arange(K, dtype=jnp.int32))
    vi_p, ri_p, ci_p = _sc_edge_index(rank, row_p, col_p)
    vi = vi_p[:E]
    ri = ri_p[:E]
    ci = ci_p[:E]
    hp = jnp.concatenate([h, pos], axis=1)[midx]
    hm = hp[:, :H]
    pm = hp[:, H:H + 3]

    hs2 = jnp.zeros((KP, S), jnp.float32).at[:K].set(hm[:, :S] @ lp['vgen_w'])
    hmt = jnp.zeros((S, KP), jnp.float32).at[:, :K].set(hm[:, :S].T)
    adj = jnp.zeros((KP, KP), jnp.float32).at[ri, ci].add(vi)
    nbr = _top8(hs2, hmt, adj)

    vr = jnp.repeat(jnp.arange(K, dtype=jnp.int32), V)
    vc = nbr.reshape(-1).astype(jnp.int32)

    a_n = hm @ lp['attn_w1'][:H]
    b_n = hm @ lp['attn_w1'][H:2 * H]
    x = hm @ lp['lin1_w']
    tab = jnp.concatenate([a_n, b_n, x, pm], axis=1)
    tv = tab[vc]
    a_vc = tv[:, :H]
    b_vc = tv[:, H:2 * H]
    x_vc = tv[:, 2 * H:3 * H]
    pm_vc = tv[:, 3 * H:3 * H + 3]

    a_rep = jnp.repeat(a_n, V, axis=0)
    b_rep = jnp.repeat(b_n, V, axis=0)
    pm_rep = jnp.repeat(pm, V, axis=0)
    ew_h = _dist(pm_rep, pm_vc)
    ew_v = jnp.concatenate([ew_h, ew_h])

    ab = jnp.concatenate([a_rep + b_vc, a_vc + b_rep]) + lp['attn_b1']
    s_v = _edge_att(ew_v, ab, lp['attn_w1'][2 * H:], lp['attn_w2']) + lp['attn_b2'][0]

    s1 = s_v[:K * V].reshape(K, V)
    s2 = s_v[K * V:]
    mx = jnp.maximum(jnp.max(s1, axis=1), jax.ops.segment_max(s2, vc, num_segments=K))
    mx = jax.lax.stop_gradient(mx)
    ex1 = jnp.exp(s1 - mx[:, None])
    ex2 = jnp.exp(s2 - mx[vc])
    den = jnp.sum(ex1, axis=1) + jax.ops.segment_sum(ex2, vc, num_segments=K)
    dec1 = ex1 / (den[:, None] + 1e-12)
    dec2 = ex2 / (den[vc] + 1e-12)
    dec_v = jnp.concatenate([dec1.reshape(-1), dec2])

    rowm = jnp.concatenate([ri, vr, vc])
    colm = jnp.concatenate([ci, vc, vr])
    ew_m = jnp.concatenate([ew_full, ew_v])
    decay = jnp.concatenate([jnp.ones((E,), jnp.float32), dec_v])
    validm = jnp.concatenate([vi, jnp.ones((2 * K * V,), jnp.float32)])
    w_m = _edge_w(ew_m, decay, validm, lp['mlp_w1'], lp['mlp_b1'], lp['mlp_w2'], lp['mlp_b2'])
    x_rows = jnp.concatenate([x[ri], jnp.repeat(x, V, axis=0), x_vc])
    agg = jnp.zeros((K, NF), jnp.float32).at[colm].add(x_rows * w_m)
    hup = _ssp(agg @ lp['lin2_w'] + lp['lin2_b']) @ lp['lin_w'] + lp['lin_b']
    hh = hm + hup
    hexp = jnp.zeros_like(h).at[midx].set(hh)
    hf = (1.0 - m[:, None]) * h + m[:, None] * hexp

    ones_e = jnp.ones((E,), jnp.float32)
    w_f = _edge_w(ew_full, ones_e, ones_e, lp['mlp_w1'], lp['mlp_b1'], lp['mlp_w2'], lp['mlp_b2'])
    x2 = hf @ lp['lin1_w']
    agg2 = jnp.zeros((N, NF), jnp.float32).at[col].add(x2[row] * w_f)
    hup2 = _ssp(agg2 @ lp['lin2_w'] + lp['lin2_b']) @ lp['lin_w'] + lp['lin_b']
    return hf + hup2


def kernel(atoms, pos, batch, edge_index, params):
    row = edge_index[0]
    col = edge_index[1]
    h = params['emb'][atoms]
    pc = pos[jnp.concatenate([row, col])]
    ew_full = _dist(pc[:E], pc[E:])
    row_p = jnp.zeros((EPAD,), jnp.int32).at[:E].set(row)
    col_p = jnp.zeros((EPAD,), jnp.int32).at[:E].set(col)
    for lp in params['layers']:
        h = _hmp(lp, h, pos, row, col, ew_full, row_p, col_p)
    h2 = jax.nn.silu(h @ params['trunk_w'] + params['trunk_b'])
    e_atom = (h2 @ params['e_w'] + params['e_b'])[:, 0]
    q_atom = (h2 @ params['q_w'] + params['q_b'])[:, 0]
    e_tot = jax.ops.segment_sum(e_atom, batch, num_segments=G)
    q_tot = jax.ops.segment_sum(q_atom, batch, num_segments=G)
    return (e_atom, q_atom, e_tot, q_tot)
